# baseline (speedup 1.0000x reference)
import functools

import jax
import jax.numpy as jnp
from jax import lax
from jax.experimental import pallas as pl
from jax.experimental.pallas import tpu as pltpu
from jax.experimental.pallas import tpu_sc as plsc

BATCH = 4096
SEQ_LEN = 200
EMBED_DIM = 64

NC = 2
NS = 16
NW = NC * NS

BBLK = BATCH // NW
K = 2
NGRP = SEQ_LEN // K


def _gather_kernel(xt_hbm, tab_hbm, out_hbm,
                   idx_v, stag0, stag1, obuf0, obuf1,
                   gsem0, gsem1, ssem0, ssem1):
    w = lax.axis_index("s") * NC + lax.axis_index("c")
    stags = (stag0, stag1)
    obufs = (obuf0, obuf1)
    gsems = (gsem0, gsem1)
    ssems = (ssem0, ssem1)

    pltpu.sync_copy(xt_hbm.at[:, pl.ds(w * BBLK, BBLK)], idx_v)

    iota = lax.iota(jnp.int32, 16)

    def issue_gathers(g, c):
        for sp in range(K):
            pltpu.async_copy(
                tab_hbm.at[idx_v.at[g * K + sp]],
                stags[c].at[pl.ds(sp * BBLK, BBLK)],
                gsems[c])

    def transpose(c):
        stag, obuf = stags[c], obufs[c]

        def rd_body(rd, carry):
            for sp in range(K):
                for dr in range(8):
                    col = jnp.full((16,), 0, jnp.int32) + (rd * 8 + dr)
                    for db0 in range(8):
                        rows = iota + (sp * BBLK + db0 * 16)
                        v = plsc.load_gather(stag, [rows, col])
                        obuf[sp, rd, dr, pl.ds(db0 * 16, 16)] = v
            return carry

        lax.fori_loop(0, 8, rd_body, 0)

    def phase(g, c):
        for sp in range(K):
            pltpu.make_async_copy(
                tab_hbm.at[idx_v.at[g * K + sp]],
                stags[c].at[pl.ds(sp * BBLK, BBLK)],
                gsems[c]).wait()

        @pl.when(g >= 2)
        def _drain_stores():
            for rd in range(8):
                pltpu.make_async_copy(
                    obufs[c].at[:, rd],
                    out_hbm.at[pl.ds((g - 2) * K, K), rd, w],
                    ssems[c]).wait()

        transpose(c)

        for rd in range(8):
            pltpu.async_copy(
                obufs[c].at[:, rd],
                out_hbm.at[pl.ds(g * K, K), rd, w],
                ssems[c])

        @pl.when(g + 2 < NGRP)
        def _refill():
            issue_gathers(g + 2, c)

    issue_gathers(0, 0)
    issue_gathers(1, 1)

    def outer(i, carry):
        phase(2 * i, 0)
        phase(2 * i + 1, 1)
        return carry

    lax.fori_loop(0, NGRP // 2, outer, 0)

    for c, g in ((0, NGRP - 2), (1, NGRP - 1)):
        for rd in range(8):
            pltpu.make_async_copy(
                obufs[c].at[:, rd],
                out_hbm.at[pl.ds(g * K, K), rd, w],
                ssems[c]).wait()


def _gather(xt, table):
    mesh = plsc.VectorSubcoreMesh(core_axis_name="c", subcore_axis_name="s")
    run = functools.partial(
        pl.kernel,
        mesh=mesh,
        compiler_params=pltpu.CompilerParams(
            use_tc_tiling_on_sc=False, needs_layout_passes=False),
        out_type=jax.ShapeDtypeStruct((SEQ_LEN, 8, NW, 8, 128), jnp.float32),
        scratch_types=[
            pltpu.VMEM((SEQ_LEN, BBLK), jnp.int32),
            pltpu.VMEM((K * BBLK, EMBED_DIM), jnp.float32),
            pltpu.VMEM((K * BBLK, EMBED_DIM), jnp.float32),
            pltpu.VMEM((K, 8, 8, 128), jnp.float32),
            pltpu.VMEM((K, 8, 8, 128), jnp.float32),
        ] + [pltpu.SemaphoreType.DMA] * 4,
    )(_gather_kernel)
    return run(xt, table)


def kernel(x, table):
    xt = jnp.transpose(x).astype(jnp.int32)
    out_phys = _gather(xt, table)
    return out_phys.transpose(2, 4, 0, 1, 3).reshape(BATCH, SEQ_LEN, EMBED_DIM)


# speedup vs baseline: 1.5551x; 1.5551x over previous
import functools

import jax
import jax.numpy as jnp
from jax import lax
from jax.experimental import pallas as pl
from jax.experimental.pallas import tpu as pltpu
from jax.experimental.pallas import tpu_sc as plsc

BATCH = 4096
SEQ_LEN = 200
EMBED_DIM = 64

NC = 2
NS = 16
NW = NC * NS

BBLK = BATCH // NW
NBUF = 8
NGRP = SEQ_LEN // NBUF


def _gather_kernel(xt_hbm, tab_hbm, out_hbm, idx_v, rows_v, *sems):
    gsems, ssems = sems[:NBUF], sems[NBUF:]
    w = lax.axis_index("s") * NC + lax.axis_index("c")
    b0 = w * BBLK

    pltpu.sync_copy(xt_hbm.at[:, pl.ds(b0, BBLK)], idx_v)

    for b in range(NBUF):
        pltpu.async_copy(tab_hbm.at[idx_v.at[b]], rows_v.at[b], gsems[b])

    def outer(g, carry):
        for b in range(NBUF):
            s = g * NBUF + b
            dst = out_hbm.at[pl.ds(b0, BBLK), s]
            pltpu.make_async_copy(
                tab_hbm.at[idx_v.at[s]], rows_v.at[b], gsems[b]).wait()
            pltpu.async_copy(rows_v.at[b], dst, ssems[b])

        @pl.when(g < NGRP - 1)
        def _refill():
            for b in range(NBUF):
                s = g * NBUF + b
                dst = out_hbm.at[pl.ds(b0, BBLK), s]
                pltpu.make_async_copy(rows_v.at[b], dst, ssems[b]).wait()
                pltpu.async_copy(
                    tab_hbm.at[idx_v.at[s + NBUF]], rows_v.at[b], gsems[b])

        return carry

    lax.fori_loop(0, NGRP, outer, 0)

    for b in range(NBUF):
        s = (NGRP - 1) * NBUF + b
        dst = out_hbm.at[pl.ds(b0, BBLK), s]
        pltpu.make_async_copy(rows_v.at[b], dst, ssems[b]).wait()


def _gather(xt, table):
    mesh = plsc.VectorSubcoreMesh(core_axis_name="c", subcore_axis_name="s")
    run = functools.partial(
        pl.kernel,
        mesh=mesh,
        compiler_params=pltpu.CompilerParams(
            use_tc_tiling_on_sc=False, needs_layout_passes=False),
        out_type=jax.ShapeDtypeStruct((BATCH, SEQ_LEN, EMBED_DIM), jnp.float32),
        scratch_types=[
            pltpu.VMEM((SEQ_LEN, BBLK), jnp.int32),
            pltpu.VMEM((NBUF, BBLK, EMBED_DIM), jnp.float32),
        ] + [pltpu.SemaphoreType.DMA] * (2 * NBUF),
    )(_gather_kernel)
    return run(xt, table)


def kernel(x, table):
    xt = jnp.transpose(x).astype(jnp.int32)
    return _gather(xt, table)
